# Initial kernel scaffold; baseline (speedup 1.0000x reference)
import jax
import jax.numpy as jnp
from jax.experimental import pallas as pl


def kernel(x, table, W, b):
    raise NotImplementedError("write your pallas kernel here")


# speedup vs baseline: 2.2526x; 2.2526x over previous
import functools

import jax
import jax.numpy as jnp
from jax import lax
from jax.experimental import pallas as pl
from jax.experimental.pallas import tpu as pltpu
from jax.experimental.pallas import tpu_sc as plsc

_BATCH = 16384
_HIST = 50
_DIM = 32
_NCLASS = 1000

_NC = 2
_NS = 16
_NW = _NC * _NS
_LANES = 16
_BAGS_PER_W = _BATCH // _NW
_CHUNK_BAGS = 2
_IDX_PER_CHUNK = _CHUNK_BAGS * _HIST
_NCHUNK = _BAGS_PER_W // _CHUNK_BAGS

_sc_mesh = plsc.VectorSubcoreMesh(core_axis_name="c", subcore_axis_name="s")


def _tree_add(vals):
    while len(vals) > 1:
        nxt = [a + b for a, b in zip(vals[0::2], vals[1::2])]
        if len(vals) % 2:
            nxt.append(vals[-1])
        vals = nxt
    return vals[0]


@functools.partial(
    pl.kernel,
    out_type=jax.ShapeDtypeStruct((_BATCH, _DIM), jnp.float32),
    mesh=_sc_mesh,
    scratch_types=[
        pltpu.VMEM((_NCHUNK, _IDX_PER_CHUNK), jnp.int32),
        pltpu.VMEM((_IDX_PER_CHUNK, _DIM), jnp.float32),
        pltpu.VMEM((_IDX_PER_CHUNK, _DIM), jnp.float32),
        pltpu.VMEM((_BAGS_PER_W, _DIM), jnp.float32),
        pltpu.SemaphoreType.DMA,
        pltpu.SemaphoreType.DMA,
    ],
    compiler_params=pltpu.CompilerParams(use_tc_tiling_on_sc=False),
)
def _bag_sum_sc(x_hbm, table_hbm, out_hbm, idx_v, g0, g1, acc_v, sem0, sem1):
    wid = lax.axis_index("c") * _NS + lax.axis_index("s")
    pltpu.sync_copy(x_hbm.at[wid], idx_v)
    pltpu.async_copy(table_hbm.at[idx_v.at[0]], g0, sem0)

    def reduce_chunk(g, chunk):
        for k in range(_CHUNK_BAGS):
            row = chunk * _CHUNK_BAGS + k
            for h in range(_DIM // _LANES):
                vals = [
                    g[k * _HIST + r, pl.ds(h * _LANES, _LANES)]
                    for r in range(_HIST)
                ]
                acc_v[row, pl.ds(h * _LANES, _LANES)] = _tree_add(vals)

    @pl.loop(0, _NCHUNK, step=2)
    def _(j):
        pltpu.async_copy(table_hbm.at[idx_v.at[j + 1]], g1, sem1)
        pltpu.make_async_copy(table_hbm.at[idx_v.at[j]], g0, sem0).wait()
        reduce_chunk(g0, j)

        @pl.when(j + 2 < _NCHUNK)
        def _():
            pltpu.async_copy(table_hbm.at[idx_v.at[j + 2]], g0, sem0)

        pltpu.make_async_copy(table_hbm.at[idx_v.at[j + 1]], g1, sem1).wait()
        reduce_chunk(g1, j + 1)

    pltpu.sync_copy(acc_v, out_hbm.at[pl.ds(wid * _BAGS_PER_W, _BAGS_PER_W)])


_BT = 512


def _mm_body(e_ref, w_ref, b_ref, o_ref):
    o_ref[...] = (
        lax.dot_general(
            e_ref[...],
            w_ref[...],
            (((1,), (0,)), ((), ())),
            preferred_element_type=jnp.float32,
            precision=lax.Precision.HIGHEST,
        )
        + b_ref[...]
    )


def _linear_tc(embed_sum, wt, b2d):
    return pl.pallas_call(
        _mm_body,
        grid=(_BATCH // _BT,),
        in_specs=[
            pl.BlockSpec((_BT, _DIM), lambda i: (i, 0)),
            pl.BlockSpec((_DIM, _NCLASS), lambda i: (0, 0)),
            pl.BlockSpec((1, _NCLASS), lambda i: (0, 0)),
        ],
        out_specs=pl.BlockSpec((_BT, _NCLASS), lambda i: (i, 0)),
        out_shape=jax.ShapeDtypeStruct((_BATCH, _NCLASS), jnp.float32),
    )(embed_sum, wt, b2d)


def kernel(x, table, W, b):
    xw = x.astype(jnp.int32).reshape(_NW, _NCHUNK, _IDX_PER_CHUNK)
    embed_sum = _bag_sum_sc(xw, table)
    wt = W.T.astype(jnp.float32) * (1.0 / _HIST)
    b2d = b.reshape(1, _NCLASS).astype(jnp.float32)
    return _linear_tc(embed_sum, wt, b2d)


# speedup vs baseline: 2.4401x; 1.0832x over previous
import functools

import jax
import jax.numpy as jnp
from jax import lax
from jax.experimental import pallas as pl
from jax.experimental.pallas import tpu as pltpu
from jax.experimental.pallas import tpu_sc as plsc

_BATCH = 16384
_HIST = 50
_DIM = 32
_NCLASS = 1000

_NC = 2
_NS = 16
_NW = _NC * _NS
_LANES = 16
_BAGS_PER_W = _BATCH // _NW
_CHUNK_BAGS = 2
_IDX_PER_CHUNK = _CHUNK_BAGS * _HIST
_NCHUNK = _BAGS_PER_W // _CHUNK_BAGS

_sc_mesh = plsc.VectorSubcoreMesh(core_axis_name="c", subcore_axis_name="s")


def _tree_add(vals):
    while len(vals) > 1:
        nxt = [a + b for a, b in zip(vals[0::2], vals[1::2])]
        if len(vals) % 2:
            nxt.append(vals[-1])
        vals = nxt
    return vals[0]


@functools.partial(
    pl.kernel,
    out_type=jax.ShapeDtypeStruct((_BATCH, _DIM), jnp.float32),
    mesh=_sc_mesh,
    scratch_types=[
        pltpu.VMEM((_NCHUNK, _IDX_PER_CHUNK), jnp.int32),
        pltpu.VMEM((_IDX_PER_CHUNK, _DIM), jnp.float32),
        pltpu.VMEM((_IDX_PER_CHUNK, _DIM), jnp.float32),
        pltpu.VMEM((_BAGS_PER_W, _DIM), jnp.float32),
        pltpu.SemaphoreType.DMA,
        pltpu.SemaphoreType.DMA,
    ],
    compiler_params=pltpu.CompilerParams(use_tc_tiling_on_sc=False),
)
def _bag_sum_sc(x_hbm, table_hbm, out_hbm, idx_v, g0, g1, acc_v, sem0, sem1):
    wid = lax.axis_index("c") * _NS + lax.axis_index("s")
    pltpu.sync_copy(x_hbm.at[wid], idx_v)
    pltpu.async_copy(table_hbm.at[idx_v.at[0]], g0, sem0)

    def reduce_chunk(g, chunk):
        for k in range(_CHUNK_BAGS):
            row = chunk * _CHUNK_BAGS + k
            for h in range(_DIM // _LANES):
                vals = [
                    g[k * _HIST + r, pl.ds(h * _LANES, _LANES)]
                    for r in range(_HIST)
                ]
                acc_v[row, pl.ds(h * _LANES, _LANES)] = _tree_add(vals)

    @pl.loop(0, _NCHUNK, step=2)
    def _(j):
        pltpu.async_copy(table_hbm.at[idx_v.at[j + 1]], g1, sem1)
        pltpu.make_async_copy(table_hbm.at[idx_v.at[j]], g0, sem0).wait()
        reduce_chunk(g0, j)

        @pl.when(j + 2 < _NCHUNK)
        def _():
            pltpu.async_copy(table_hbm.at[idx_v.at[j + 2]], g0, sem0)

        pltpu.make_async_copy(table_hbm.at[idx_v.at[j + 1]], g1, sem1).wait()
        reduce_chunk(g1, j + 1)

    pltpu.sync_copy(acc_v, out_hbm.at[pl.ds(wid * _BAGS_PER_W, _BAGS_PER_W)])


_TCW = 1024


def _tp_body(i_ref, o_ref):
    t = i_ref[...]
    tt = jnp.transpose(t)
    o_ref[...] = jnp.concatenate([tt[k::4, :] for k in range(4)], axis=1)


def _table_rowmajor_tc(tT):
    n = tT.shape[1]
    grid = (n + _TCW - 1) // _TCW
    return pl.pallas_call(
        _tp_body,
        grid=(grid,),
        in_specs=[pl.BlockSpec((_DIM, _TCW), lambda i: (0, i))],
        out_specs=pl.BlockSpec((_TCW * _DIM // 128, 128), lambda i: (i, 0)),
        out_shape=jax.ShapeDtypeStruct((n * _DIM // 128, 128), jnp.float32),
    )(tT)


_BT = 512


def _mm_body(w_ref, e_ref, b_ref, o_ref):
    o_ref[...] = (
        lax.dot_general(
            w_ref[...],
            e_ref[...],
            (((1,), (1,)), ((), ())),
            preferred_element_type=jnp.float32,
            precision=lax.Precision.HIGHEST,
        )
        + b_ref[...]
    )


def _linear_tc(embed_sum, wt, bcol):
    return pl.pallas_call(
        _mm_body,
        grid=(_BATCH // _BT,),
        in_specs=[
            pl.BlockSpec((_NCLASS, _DIM), lambda i: (0, 0)),
            pl.BlockSpec((_BT, _DIM), lambda i: (i, 0)),
            pl.BlockSpec((_NCLASS, 1), lambda i: (0, 0)),
        ],
        out_specs=pl.BlockSpec((_NCLASS, _BT), lambda i: (0, i)),
        out_shape=jax.ShapeDtypeStruct((_NCLASS, _BATCH), jnp.float32),
    )(wt, embed_sum, bcol)


def kernel(x, table, W, b):
    xw = x.astype(jnp.int32).reshape(_NW, _NCHUNK, _IDX_PER_CHUNK)
    n_rows = table.shape[0]
    t250 = lax.optimization_barrier(jnp.reshape(table, (n_rows * _DIM // 128, 128)))
    tbl = jnp.reshape(t250, (n_rows, _DIM))
    embed_sum = _bag_sum_sc(xw, tbl)
    wt = W.astype(jnp.float32) * (1.0 / _HIST)
    bcol = b.reshape(_NCLASS, 1).astype(jnp.float32)
    out_t = _linear_tc(embed_sum, wt, bcol)
    return jnp.transpose(out_t)
